# baseline (speedup 1.0000x reference)
import functools

import jax
import jax.numpy as jnp
from jax import lax
from jax.experimental import pallas as pl
from jax.experimental.pallas import tpu as pltpu
from jax.experimental.pallas import tpu_sc as plsc

BATCH = 16384
EMB_DIM = 64
NUM_WORKERS = 32
PER_W = BATCH // NUM_WORKERS
CHUNK = 128
N_CHUNKS = PER_W // CHUNK


def _body(bh_hbm, bt_hbm, br_hbm, ent_hbm, rel_hbm, out_hbm,
          idx_h, idx_t, idx_r, h_rows, t_rows, r_rows, out_v, sem):
    wid = lax.axis_index("c") * 16 + lax.axis_index("s")
    row0 = wid * N_CHUNKS

    pltpu.sync_copy(bh_hbm.at[pl.ds(row0, N_CHUNKS)], idx_h)
    pltpu.sync_copy(bt_hbm.at[pl.ds(row0, N_CHUNKS)], idx_t)
    pltpu.sync_copy(br_hbm.at[pl.ds(row0, N_CHUNKS)], idx_r)

    copies = []
    for c in range(N_CHUNKS):
        dst = pl.ds(c * CHUNK, CHUNK)
        copies.append(pltpu.async_copy(ent_hbm.at[idx_h.at[c]], h_rows.at[dst], sem))
        copies.append(pltpu.async_copy(ent_hbm.at[idx_t.at[c]], t_rows.at[dst], sem))
        copies.append(pltpu.async_copy(rel_hbm.at[idx_r.at[c]], r_rows.at[dst], sem))
    for cp in copies:
        cp.wait()

    lanes = lax.broadcasted_iota(jnp.int32, (16,), 0)

    def group(g, _):
        rows16 = g * 16 + lanes

        def dstep(d, acc):
            cols = lax.bitwise_and(d + lanes, EMB_DIM - 1)
            h = plsc.load_gather(h_rows, [rows16, cols])
            t = plsc.load_gather(t_rows, [rows16, cols])
            r = plsc.load_gather(r_rows, [rows16, cols])
            return acc + h * r * t

        acc = lax.fori_loop(0, EMB_DIM, dstep, jnp.zeros((16,), jnp.float32))
        out_v[pl.ds(g * 16, 16)] = 1.0 / (1.0 + jnp.exp(-acc))
        return 0

    lax.fori_loop(0, PER_W // 16, group, 0)
    pltpu.sync_copy(out_v, out_hbm.at[pl.ds(wid * PER_W, PER_W)])


@functools.partial(jax.jit, static_argnames=())
def _dist_mul(bh, bt, br, ent_emb, rel_emb):
    mesh = plsc.VectorSubcoreMesh(core_axis_name="c", subcore_axis_name="s")
    kern = functools.partial(
        pl.kernel,
        out_type=jax.ShapeDtypeStruct((BATCH,), jnp.float32),
        mesh=mesh,
        scratch_types=[
            pltpu.VMEM((N_CHUNKS, CHUNK), jnp.int32),
            pltpu.VMEM((N_CHUNKS, CHUNK), jnp.int32),
            pltpu.VMEM((N_CHUNKS, CHUNK), jnp.int32),
            pltpu.VMEM((PER_W, EMB_DIM), jnp.float32),
            pltpu.VMEM((PER_W, EMB_DIM), jnp.float32),
            pltpu.VMEM((PER_W, EMB_DIM), jnp.float32),
            pltpu.VMEM((PER_W,), jnp.float32),
            pltpu.SemaphoreType.DMA,
        ],
        compiler_params=pltpu.CompilerParams(
            use_tc_tiling_on_sc=False, needs_layout_passes=False
        ),
    )(_body)
    return kern(bh, bt, br, ent_emb, rel_emb)


def kernel(batch_h, batch_t, batch_r, ent_emb, rel_emb):
    bh = batch_h.astype(jnp.int32).reshape(BATCH // CHUNK, CHUNK)
    bt = batch_t.astype(jnp.int32).reshape(BATCH // CHUNK, CHUNK)
    br = batch_r.astype(jnp.int32).reshape(BATCH // CHUNK, CHUNK)
    return _dist_mul(bh, bt, br, ent_emb, rel_emb)


# speedup vs baseline: 1.5970x; 1.5970x over previous
import functools

import jax
import jax.numpy as jnp
from jax import lax
from jax.experimental import pallas as pl
from jax.experimental.pallas import tpu as pltpu
from jax.experimental.pallas import tpu_sc as plsc

BATCH = 16384
EMB_DIM = 64
NUM_WORKERS = 32
PER_W = BATCH // NUM_WORKERS
GROUPS = PER_W // 16


def _body(bh_hbm, bt_hbm, br_hbm, ent_hbm, rel_hbm, out_hbm,
          idx_h, idx_t, idx_r, hbuf, tbuf, rbuf, out_v, sem):
    wid = lax.axis_index("c") * 16 + lax.axis_index("s")
    row0 = wid * 4

    pltpu.sync_copy(bh_hbm.at[pl.ds(row0, 4)], idx_h)
    pltpu.sync_copy(bt_hbm.at[pl.ds(row0, 4)], idx_t)
    pltpu.sync_copy(br_hbm.at[pl.ds(row0, 4)], idx_r)

    lanes = lax.broadcasted_iota(jnp.int32, (16,), 0)

    def group(g, _):
        c = g // 8
        j = (g % 8) * 16
        hvec = idx_h[c, pl.ds(j, 16)]
        tvec = idx_t[c, pl.ds(j, 16)]
        rvec = idx_r[c, pl.ds(j, 16)]
        copies = []
        for k in range(16):
            dst = pl.ds(k, 1)
            copies.append(pltpu.async_copy(
                ent_hbm.at[pl.ds(hvec[k], 1)], hbuf.at[dst], sem))
            copies.append(pltpu.async_copy(
                ent_hbm.at[pl.ds(tvec[k], 1)], tbuf.at[dst], sem))
            copies.append(pltpu.async_copy(
                rel_hbm.at[pl.ds(rvec[k], 1)], rbuf.at[dst], sem))
        for cp in copies:
            cp.wait()

        def dstep(d, acc):
            cols = lax.bitwise_and(d + lanes, EMB_DIM - 1)
            h = plsc.load_gather(hbuf, [lanes, cols])
            t = plsc.load_gather(tbuf, [lanes, cols])
            r = plsc.load_gather(rbuf, [lanes, cols])
            return acc + h * r * t

        acc = lax.fori_loop(0, EMB_DIM, dstep, jnp.zeros((16,), jnp.float32))
        out_v[pl.ds(g * 16, 16)] = 1.0 / (1.0 + jnp.exp(-acc))
        return 0

    lax.fori_loop(0, GROUPS, group, 0)
    pltpu.sync_copy(out_v, out_hbm.at[pl.ds(wid * PER_W, PER_W)])


@jax.jit
def _dist_mul(bh, bt, br, ent_emb, rel_emb):
    mesh = plsc.VectorSubcoreMesh(core_axis_name="c", subcore_axis_name="s")
    kern = functools.partial(
        pl.kernel,
        out_type=jax.ShapeDtypeStruct((BATCH,), jnp.float32),
        mesh=mesh,
        scratch_types=[
            pltpu.VMEM((4, 128), jnp.int32),
            pltpu.VMEM((4, 128), jnp.int32),
            pltpu.VMEM((4, 128), jnp.int32),
            pltpu.VMEM((16, EMB_DIM), jnp.float32),
            pltpu.VMEM((16, EMB_DIM), jnp.float32),
            pltpu.VMEM((16, EMB_DIM), jnp.float32),
            pltpu.VMEM((PER_W,), jnp.float32),
            pltpu.SemaphoreType.DMA,
        ],
        compiler_params=pltpu.CompilerParams(needs_layout_passes=False),
    )(_body)
    return kern(bh, bt, br, ent_emb, rel_emb)


def kernel(batch_h, batch_t, batch_r, ent_emb, rel_emb):
    bh = batch_h.astype(jnp.int32).reshape(128, 128)
    bt = batch_t.astype(jnp.int32).reshape(128, 128)
    br = batch_r.astype(jnp.int32).reshape(128, 128)
    return _dist_mul(bh, bt, br, ent_emb, rel_emb)


# speedup vs baseline: 1.6884x; 1.0573x over previous
import functools

import jax
import jax.numpy as jnp
from jax import lax
from jax.experimental import pallas as pl
from jax.experimental.pallas import tpu as pltpu
from jax.experimental.pallas import tpu_sc as plsc

BATCH = 16384
EMB_DIM = 64
NUM_WORKERS = 32
PER_W = BATCH // NUM_WORKERS
GROUPS = PER_W // 16


PASS = 128
N_PASS = PER_W // PASS
GP = PASS // 16


def _body(bh_hbm, bt_hbm, br_hbm, ent_hbm, rel_hbm, out_hbm,
          idx_h, idx_t, idx_r,
          hbuf0, tbuf0, rbuf0, hbuf1, tbuf1, rbuf1, out_v, sem0, sem1):
    wid = lax.axis_index("c") * 16 + lax.axis_index("s")
    row0 = wid * 4

    pltpu.sync_copy(bh_hbm.at[pl.ds(row0, 4)], idx_h)
    pltpu.sync_copy(bt_hbm.at[pl.ds(row0, 4)], idx_t)
    pltpu.sync_copy(br_hbm.at[pl.ds(row0, 4)], idx_r)

    lanes = lax.broadcasted_iota(jnp.int32, (16,), 0)
    bufs = ((hbuf0, tbuf0, rbuf0), (hbuf1, tbuf1, rbuf1))
    sems = (sem0, sem1)

    def fire(p, hb, tb, rb, sem):
        def fire_group(gl, _):
            j = gl * 16
            hvec = idx_h[p, pl.ds(j, 16)]
            tvec = idx_t[p, pl.ds(j, 16)]
            rvec = idx_r[p, pl.ds(j, 16)]
            for k in range(16):
                dst = pl.ds(j + k, 1)
                pltpu.async_copy(ent_hbm.at[pl.ds(hvec[k], 1)], hb.at[dst], sem)
                pltpu.async_copy(ent_hbm.at[pl.ds(tvec[k], 1)], tb.at[dst], sem)
                pltpu.async_copy(rel_hbm.at[pl.ds(rvec[k], 1)], rb.at[dst], sem)
            return 0
        lax.fori_loop(0, GP, fire_group, 0)

    def drain(hb, tb, rb, sem):
        pltpu.make_async_copy(ent_hbm.at[pl.ds(0, PASS)], hb, sem).wait()
        pltpu.make_async_copy(ent_hbm.at[pl.ds(0, PASS)], tb, sem).wait()
        pltpu.make_async_copy(ent_hbm.at[pl.ds(0, PASS)], rb, sem).wait()

    def compute(p, hb, tb, rb):
        def group(gl, _):
            rows16 = gl * 16 + lanes

            def dstep(d, acc):
                cols = lax.bitwise_and(d + lanes, EMB_DIM - 1)
                h = plsc.load_gather(hb, [rows16, cols])
                t = plsc.load_gather(tb, [rows16, cols])
                r = plsc.load_gather(rb, [rows16, cols])
                return acc + h * r * t

            acc = lax.fori_loop(0, EMB_DIM, dstep,
                                jnp.zeros((16,), jnp.float32))
            out_v[pl.ds(p * PASS + gl * 16, 16)] = 1.0 / (1.0 + jnp.exp(-acc))
            return 0
        lax.fori_loop(0, GP, group, 0)

    fire(0, *bufs[0], sems[0])
    for p in range(N_PASS):
        par = p % 2
        if p + 1 < N_PASS:
            fire(p + 1, *bufs[1 - par], sems[1 - par])
        drain(*bufs[par], sems[par])
        compute(p, *bufs[par])

    pltpu.sync_copy(out_v, out_hbm.at[pl.ds(wid * PER_W, PER_W)])


@jax.jit
def _dist_mul(bh, bt, br, ent_emb, rel_emb):
    mesh = plsc.VectorSubcoreMesh(core_axis_name="c", subcore_axis_name="s")
    kern = functools.partial(
        pl.kernel,
        out_type=jax.ShapeDtypeStruct((BATCH,), jnp.float32),
        mesh=mesh,
        scratch_types=[
            pltpu.VMEM((4, 128), jnp.int32),
            pltpu.VMEM((4, 128), jnp.int32),
            pltpu.VMEM((4, 128), jnp.int32),
            pltpu.VMEM((PASS, EMB_DIM), jnp.float32),
            pltpu.VMEM((PASS, EMB_DIM), jnp.float32),
            pltpu.VMEM((PASS, EMB_DIM), jnp.float32),
            pltpu.VMEM((PASS, EMB_DIM), jnp.float32),
            pltpu.VMEM((PASS, EMB_DIM), jnp.float32),
            pltpu.VMEM((PASS, EMB_DIM), jnp.float32),
            pltpu.VMEM((PER_W,), jnp.float32),
            pltpu.SemaphoreType.DMA,
            pltpu.SemaphoreType.DMA,
        ],
        compiler_params=pltpu.CompilerParams(needs_layout_passes=False),
    )(_body)
    return kern(bh, bt, br, ent_emb, rel_emb)


def kernel(batch_h, batch_t, batch_r, ent_emb, rel_emb):
    bh = batch_h.astype(jnp.int32).reshape(128, 128)
    bt = batch_t.astype(jnp.int32).reshape(128, 128)
    br = batch_r.astype(jnp.int32).reshape(128, 128)
    return _dist_mul(bh, bt, br, ent_emb, rel_emb)


# speedup vs baseline: 2.4940x; 1.4771x over previous
import functools

import jax
import jax.numpy as jnp
from jax import lax
from jax.experimental import pallas as pl
from jax.experimental.pallas import tpu as pltpu
from jax.experimental.pallas import tpu_sc as plsc

BATCH = 16384
EMB_DIM = 64
NUM_WORKERS = 32
PER_W = BATCH // NUM_WORKERS
PASS = 128
N_PASS = PER_W // PASS
GP = PASS // 16


def _body(bh_hbm, bt_hbm, br_hbm, ent_hbm, rel_hbm, out_hbm,
          idx_h, idx_t, idx_r,
          hbuf0, tbuf0, rbuf0, hbuf1, tbuf1, rbuf1, out_v, sem0, sem1):
    wid = lax.axis_index("c") * 16 + lax.axis_index("s")
    row0 = wid * N_PASS

    pltpu.sync_copy(bh_hbm.at[pl.ds(row0, N_PASS)], idx_h)
    pltpu.sync_copy(bt_hbm.at[pl.ds(row0, N_PASS)], idx_t)
    pltpu.sync_copy(br_hbm.at[pl.ds(row0, N_PASS)], idx_r)

    lanes = lax.broadcasted_iota(jnp.int32, (16,), 0)
    bufs = ((hbuf0, tbuf0, rbuf0), (hbuf1, tbuf1, rbuf1))
    sems = (sem0, sem1)

    def row_dma(table, i, buf, k, sem):
        src = table.at[lax.shift_right_logical(i, 3), pl.ds(lax.bitwise_and(i, 7), 1)]
        pltpu.async_copy(src, buf.at[pl.ds(k, 1)], sem)

    def fire(p, hb, tb, rb, sem):
        def fire_group(gl, _):
            j = gl * 16
            hvec = idx_h[p, pl.ds(j, 16)]
            tvec = idx_t[p, pl.ds(j, 16)]
            rvec = idx_r[p, pl.ds(j, 16)]
            for k in range(16):
                row_dma(ent_hbm, hvec[k], hb, j + k, sem)
                row_dma(ent_hbm, tvec[k], tb, j + k, sem)
                row_dma(rel_hbm, rvec[k], rb, j + k, sem)
            return 0
        lax.fori_loop(0, GP, fire_group, 0)

    def drain(hb, tb, rb, sem):
        dummy = ent_hbm.at[pl.ds(0, PASS), 0]
        pltpu.make_async_copy(dummy, hb, sem).wait()
        pltpu.make_async_copy(dummy, tb, sem).wait()
        pltpu.make_async_copy(dummy, rb, sem).wait()

    def compute(p, hb, tb, rb):
        def group(gl, _):
            rows16 = gl * 16 + lanes

            def dstep(d, acc):
                cols = lax.bitwise_and(d + lanes, EMB_DIM - 1)
                h = plsc.load_gather(hb, [rows16, cols])
                t = plsc.load_gather(tb, [rows16, cols])
                r = plsc.load_gather(rb, [rows16, cols])
                return acc + h * r * t

            acc = lax.fori_loop(0, EMB_DIM, dstep,
                                jnp.zeros((16,), jnp.float32))
            out_v[pl.ds(p * PASS + gl * 16, 16)] = 1.0 / (1.0 + jnp.exp(-acc))
            return 0
        lax.fori_loop(0, GP, group, 0)

    fire(0, *bufs[0], sems[0])
    for p in range(N_PASS):
        par = p % 2
        if p + 1 < N_PASS:
            fire(p + 1, *bufs[1 - par], sems[1 - par])
        drain(*bufs[par], sems[par])
        compute(p, *bufs[par])

    pltpu.sync_copy(out_v, out_hbm.at[pl.ds(wid * PER_W, PER_W)])


@jax.jit
def _dist_mul(bh, bt, br, ent_emb, rel_emb):
    mesh = plsc.VectorSubcoreMesh(core_axis_name="c", subcore_axis_name="s")
    kern = functools.partial(
        pl.kernel,
        out_type=jax.ShapeDtypeStruct((BATCH,), jnp.float32),
        mesh=mesh,
        scratch_types=[
            pltpu.VMEM((N_PASS, PASS), jnp.int32),
            pltpu.VMEM((N_PASS, PASS), jnp.int32),
            pltpu.VMEM((N_PASS, PASS), jnp.int32),
            pltpu.VMEM((PASS, EMB_DIM), jnp.float32),
            pltpu.VMEM((PASS, EMB_DIM), jnp.float32),
            pltpu.VMEM((PASS, EMB_DIM), jnp.float32),
            pltpu.VMEM((PASS, EMB_DIM), jnp.float32),
            pltpu.VMEM((PASS, EMB_DIM), jnp.float32),
            pltpu.VMEM((PASS, EMB_DIM), jnp.float32),
            pltpu.VMEM((PER_W,), jnp.float32),
            pltpu.SemaphoreType.DMA,
            pltpu.SemaphoreType.DMA,
        ],
        compiler_params=pltpu.CompilerParams(needs_layout_passes=False),
    )(_body)
    return kern(bh, bt, br, ent_emb, rel_emb)


def kernel(batch_h, batch_t, batch_r, ent_emb, rel_emb):
    bh = batch_h.astype(jnp.int32).reshape(128, 128)
    bt = batch_t.astype(jnp.int32).reshape(128, 128)
    br = batch_r.astype(jnp.int32).reshape(128, 128)
    ent3 = ent_emb.reshape(ent_emb.shape[0] // 8, 8, EMB_DIM)
    rel3 = rel_emb.reshape(rel_emb.shape[0] // 8, 8, EMB_DIM)
    return _dist_mul(bh, bt, br, ent3, rel3)


# speedup vs baseline: 2.4950x; 1.0004x over previous
import functools

import jax
import jax.numpy as jnp
from jax import lax
from jax.experimental import pallas as pl
from jax.experimental.pallas import tpu as pltpu
from jax.experimental.pallas import tpu_sc as plsc

BATCH = 16384
EMB_DIM = 64
NUM_WORKERS = 32
PER_W = BATCH // NUM_WORKERS
PASS = 128
N_PASS = PER_W // PASS
GP = PASS // 16


def _body(bh_hbm, bt_hbm, br_hbm, ent_hbm, rel_hbm, out_hbm,
          idx_h, idx_t, idx_r,
          hbuf0, tbuf0, rbuf0, hbuf1, tbuf1, rbuf1, out_v, sem0, sem1):
    wid = lax.axis_index("c") * 16 + lax.axis_index("s")
    row0 = wid * N_PASS

    pltpu.sync_copy(bh_hbm.at[pl.ds(row0, N_PASS)], idx_h)
    pltpu.sync_copy(bt_hbm.at[pl.ds(row0, N_PASS)], idx_t)
    pltpu.sync_copy(br_hbm.at[pl.ds(row0, N_PASS)], idx_r)

    lanes = lax.broadcasted_iota(jnp.int32, (16,), 0)
    bufs = ((hbuf0, tbuf0, rbuf0), (hbuf1, tbuf1, rbuf1))
    sems = (sem0, sem1)

    def row_dma(table, i, buf, k, sem):
        src = table.at[lax.shift_right_logical(i, 3), pl.ds(lax.bitwise_and(i, 7), 1)]
        pltpu.async_copy(src, buf.at[pl.ds(k, 1)], sem)

    def fire(p, hb, tb, rb, sem):
        def fire_group(gl, _):
            j = gl * 16
            hvec = idx_h[p, pl.ds(j, 16)]
            tvec = idx_t[p, pl.ds(j, 16)]
            rvec = idx_r[p, pl.ds(j, 16)]
            for k in range(16):
                row_dma(ent_hbm, hvec[k], hb, j + k, sem)
                row_dma(ent_hbm, tvec[k], tb, j + k, sem)
                row_dma(rel_hbm, rvec[k], rb, j + k, sem)
            return 0
        lax.fori_loop(0, GP, fire_group, 0)

    def drain(hb, tb, rb, sem):
        dummy = ent_hbm.at[pl.ds(0, PASS), 0]
        pltpu.make_async_copy(dummy, hb, sem).wait()
        pltpu.make_async_copy(dummy, tb, sem).wait()
        pltpu.make_async_copy(dummy, rb, sem).wait()

    def compute(p, hb, tb, rb):
        def group(gl, _):
            rows16 = gl * 16 + lanes

            def dstep(d, acc):
                cols = lax.bitwise_and(d + lanes, EMB_DIM - 1)
                h = plsc.load_gather(hb, [rows16, cols])
                t = plsc.load_gather(tb, [rows16, cols])
                r = plsc.load_gather(rb, [rows16, cols])
                return acc + h * r * t

            acc = lax.fori_loop(0, EMB_DIM, dstep,
                                jnp.zeros((16,), jnp.float32))
            out_v[pl.ds(p * PASS + gl * 16, 16)] = 1.0 / (1.0 + jnp.exp(-acc))
            return 0
        lax.fori_loop(0, GP, group, 0)

    fire(0, *bufs[0], sems[0])
    for p in range(N_PASS):
        par = p % 2
        if p + 1 < N_PASS:
            fire(p + 1, *bufs[1 - par], sems[1 - par])
        drain(*bufs[par], sems[par])
        compute(p, *bufs[par])

    pltpu.sync_copy(out_v, out_hbm.at[pl.ds(wid * PER_W, PER_W)])


@jax.jit
def _dist_mul(bh, bt, br, ent_emb, rel_emb):
    mesh = plsc.VectorSubcoreMesh(core_axis_name="c", subcore_axis_name="s")
    kern = functools.partial(
        pl.kernel,
        out_type=jax.ShapeDtypeStruct((BATCH,), jnp.float32),
        mesh=mesh,
        scratch_types=[
            pltpu.VMEM((N_PASS, PASS), jnp.int32),
            pltpu.VMEM((N_PASS, PASS), jnp.int32),
            pltpu.VMEM((N_PASS, PASS), jnp.int32),
            pltpu.VMEM((PASS, EMB_DIM), jnp.float32),
            pltpu.VMEM((PASS, EMB_DIM), jnp.float32),
            pltpu.VMEM((PASS, EMB_DIM), jnp.float32),
            pltpu.VMEM((PASS, EMB_DIM), jnp.float32),
            pltpu.VMEM((PASS, EMB_DIM), jnp.float32),
            pltpu.VMEM((PASS, EMB_DIM), jnp.float32),
            pltpu.VMEM((PER_W,), jnp.float32),
            pltpu.SemaphoreType.DMA,
            pltpu.SemaphoreType.DMA,
        ],
        compiler_params=pltpu.CompilerParams(
            needs_layout_passes=False,
            skip_device_barrier=True,
            disable_bounds_checks=True,
        ),
    )(_body)
    return kern(bh, bt, br, ent_emb, rel_emb)


def kernel(batch_h, batch_t, batch_r, ent_emb, rel_emb):
    bh = batch_h.astype(jnp.int32).reshape(128, 128)
    bt = batch_t.astype(jnp.int32).reshape(128, 128)
    br = batch_r.astype(jnp.int32).reshape(128, 128)
    ent3 = ent_emb.reshape(ent_emb.shape[0] // 8, 8, EMB_DIM)
    rel3 = rel_emb.reshape(rel_emb.shape[0] // 8, 8, EMB_DIM)
    return _dist_mul(bh, bt, br, ent3, rel3)


# speedup vs baseline: 2.5342x; 1.0157x over previous
import functools

import jax
import jax.numpy as jnp
from jax import lax
from jax.experimental import pallas as pl
from jax.experimental.pallas import tpu as pltpu
from jax.experimental.pallas import tpu_sc as plsc

BATCH = 16384
EMB_DIM = 64
NUM_WORKERS = 32
PER_W = BATCH // NUM_WORKERS
PASS = 128
N_PASS = PER_W // PASS
GP = PASS // 16


def _body(bh_hbm, bt_hbm, br_hbm, ent_hbm, rel_hbm, out_hbm,
          idx_h, idx_t, idx_r,
          hbuf0, tbuf0, rbuf0, hbuf1, tbuf1, rbuf1, out_v, sem0, sem1):
    wid = lax.axis_index("c") * 16 + lax.axis_index("s")
    row0 = wid * N_PASS

    pltpu.sync_copy(bh_hbm.at[pl.ds(row0, N_PASS)], idx_h)
    pltpu.sync_copy(bt_hbm.at[pl.ds(row0, N_PASS)], idx_t)
    pltpu.sync_copy(br_hbm.at[pl.ds(row0, N_PASS)], idx_r)

    lanes = lax.broadcasted_iota(jnp.int32, (16,), 0)
    bufs = ((hbuf0, tbuf0, rbuf0), (hbuf1, tbuf1, rbuf1))
    sems = (sem0, sem1)

    def row_dma(table, i, buf, k, sem):
        src = table.at[lax.shift_right_logical(i, 3), pl.ds(lax.bitwise_and(i, 7), 1)]
        pltpu.async_copy(src, buf.at[pl.ds(k, 1)], sem)

    def fire(p, hb, tb, rb, sem):
        def fire_group(gl, _):
            j = gl * 16
            hvec = idx_h[p, pl.ds(j, 16)]
            tvec = idx_t[p, pl.ds(j, 16)]
            rvec = idx_r[p, pl.ds(j, 16)]
            for k in range(16):
                row_dma(ent_hbm, hvec[k], hb, j + k, sem)
                row_dma(ent_hbm, tvec[k], tb, j + k, sem)
                row_dma(rel_hbm, rvec[k], rb, j + k, sem)
            return 0
        lax.fori_loop(0, GP, fire_group, 0)

    def drain(hb, tb, rb, sem):
        dummy = ent_hbm.at[pl.ds(0, PASS), 0]
        pltpu.make_async_copy(dummy, hb, sem).wait()
        pltpu.make_async_copy(dummy, tb, sem).wait()
        pltpu.make_async_copy(dummy, rb, sem).wait()

    def compute(p, hb, tb, rb):
        def group(gl, _):
            rows16 = gl * 16 + lanes

            def dstep(d, acc):
                cols = lax.bitwise_and(d + lanes, EMB_DIM - 1)
                h = plsc.load_gather(hb, [rows16, cols])
                t = plsc.load_gather(tb, [rows16, cols])
                r = plsc.load_gather(rb, [rows16, cols])
                return acc + h * r * t

            acc = lax.fori_loop(0, EMB_DIM, dstep,
                                jnp.zeros((16,), jnp.float32), unroll=4)
            out_v[pl.ds(p * PASS + gl * 16, 16)] = 1.0 / (1.0 + jnp.exp(-acc))
            return 0
        lax.fori_loop(0, GP, group, 0)

    fire(0, *bufs[0], sems[0])
    for p in range(N_PASS):
        par = p % 2
        if p + 1 < N_PASS:
            fire(p + 1, *bufs[1 - par], sems[1 - par])
        drain(*bufs[par], sems[par])
        compute(p, *bufs[par])

    pltpu.sync_copy(out_v, out_hbm.at[pl.ds(wid * PER_W, PER_W)])


@jax.jit
def _dist_mul(bh, bt, br, ent_emb, rel_emb):
    mesh = plsc.VectorSubcoreMesh(core_axis_name="c", subcore_axis_name="s")
    kern = functools.partial(
        pl.kernel,
        out_type=jax.ShapeDtypeStruct((BATCH,), jnp.float32),
        mesh=mesh,
        scratch_types=[
            pltpu.VMEM((N_PASS, PASS), jnp.int32),
            pltpu.VMEM((N_PASS, PASS), jnp.int32),
            pltpu.VMEM((N_PASS, PASS), jnp.int32),
            pltpu.VMEM((PASS, EMB_DIM), jnp.float32),
            pltpu.VMEM((PASS, EMB_DIM), jnp.float32),
            pltpu.VMEM((PASS, EMB_DIM), jnp.float32),
            pltpu.VMEM((PASS, EMB_DIM), jnp.float32),
            pltpu.VMEM((PASS, EMB_DIM), jnp.float32),
            pltpu.VMEM((PASS, EMB_DIM), jnp.float32),
            pltpu.VMEM((PER_W,), jnp.float32),
            pltpu.SemaphoreType.DMA,
            pltpu.SemaphoreType.DMA,
        ],
        compiler_params=pltpu.CompilerParams(
            needs_layout_passes=False,
            skip_device_barrier=True,
            disable_bounds_checks=True,
        ),
    )(_body)
    return kern(bh, bt, br, ent_emb, rel_emb)


def kernel(batch_h, batch_t, batch_r, ent_emb, rel_emb):
    bh = batch_h.astype(jnp.int32).reshape(128, 128)
    bt = batch_t.astype(jnp.int32).reshape(128, 128)
    br = batch_r.astype(jnp.int32).reshape(128, 128)
    ent3 = ent_emb.reshape(ent_emb.shape[0] // 8, 8, EMB_DIM)
    rel3 = rel_emb.reshape(rel_emb.shape[0] // 8, 8, EMB_DIM)
    return _dist_mul(bh, bt, br, ent3, rel3)
